# baseline (speedup 1.0000x reference)
import jax
import jax.numpy as jnp
from jax import lax
from jax.experimental import pallas as pl
from jax.experimental.pallas import tpu as pltpu


def kernel(x, embed_weight):
    B, W, F = x.shape
    emb = embed_weight[:W]
    NBUF = 4
    T = B // NBUF

    def body(x_hbm, emb_hbm, o_hbm, emb_v, ibuf, obuf, sem_e, *sems):
        sem_in = sems[:NBUF]
        sem_out = sems[NBUF:]

        pltpu.make_async_copy(emb_hbm, emb_v, sem_e).start()
        pltpu.make_async_copy(emb_hbm, emb_v, sem_e).wait()

        def start_in(m, k):
            pltpu.make_async_copy(x_hbm.at[m], ibuf.at[k], sem_in[k]).start()

        def wait_in(k):
            pltpu.make_async_copy(x_hbm.at[0], ibuf.at[k], sem_in[k]).wait()

        def start_out(m, k):
            pltpu.make_async_copy(obuf.at[k], o_hbm.at[m], sem_out[k]).start()

        def wait_out(k):
            pltpu.make_async_copy(obuf.at[k], o_hbm.at[0], sem_out[k]).wait()

        for k in range(NBUF):
            start_in(k, k)

        def lbody(t, carry):
            for k in range(NBUF):
                m = t * NBUF + k
                wait_in(k)

                @pl.when(t >= 1)
                def _():
                    wait_out(k)

                obuf[k] = ibuf[k] + emb_v[...]
                start_out(m, k)

                @pl.when(t + 1 < T)
                def _():
                    start_in(m + NBUF, k)
            return carry

        lax.fori_loop(0, T, lbody, 0)
        for k in range(NBUF):
            wait_out(k)

    return pl.pallas_call(
        body,
        in_specs=[
            pl.BlockSpec(memory_space=pl.ANY),
            pl.BlockSpec(memory_space=pl.ANY),
        ],
        out_specs=pl.BlockSpec(memory_space=pl.ANY),
        out_shape=jax.ShapeDtypeStruct(x.shape, x.dtype),
        scratch_shapes=[
            pltpu.VMEM((W, F), x.dtype),
            pltpu.VMEM((NBUF, W, F), x.dtype),
            pltpu.VMEM((NBUF, W, F), x.dtype),
            pltpu.SemaphoreType.DMA,
        ]
        + [pltpu.SemaphoreType.DMA] * (2 * NBUF),
        compiler_params=pltpu.CompilerParams(
            vmem_limit_bytes=100 * 1024 * 1024,
        ),
    )(x, emb)


# speedup vs baseline: 1.0076x; 1.0076x over previous
import jax
import jax.numpy as jnp
from jax.experimental import pallas as pl
from jax.experimental.pallas import tpu as pltpu


def _add_block(x_ref, emb_ref, o_ref):
    o_ref[...] = x_ref[...] + emb_ref[...]


def kernel(x, embed_weight):
    B, W, F = x.shape
    emb = embed_weight[:W]
    BB = 2
    return pl.pallas_call(
        _add_block,
        grid=(B // BB,),
        in_specs=[
            pl.BlockSpec((BB, W, F), lambda b: (b, 0, 0)),
            pl.BlockSpec((W, F), lambda b: (0, 0)),
        ],
        out_specs=pl.BlockSpec((BB, W, F), lambda b: (b, 0, 0)),
        out_shape=jax.ShapeDtypeStruct(x.shape, x.dtype),
        compiler_params=pltpu.CompilerParams(
            vmem_limit_bytes=100 * 1024 * 1024,
        ),
    )(x, emb)


# speedup vs baseline: 1.0076x; 1.0000x over previous
import jax
import jax.numpy as jnp
from jax.experimental import pallas as pl
from jax.experimental.pallas import tpu as pltpu


def _add_block(x_ref, emb_ref, o_ref):
    o_ref[...] = x_ref[...] + emb_ref[...]


def kernel(x, embed_weight):
    B, W, F = x.shape
    emb = embed_weight[:W]
    BB = 2
    return pl.pallas_call(
        _add_block,
        grid=(B // BB,),
        in_specs=[
            pl.BlockSpec((BB, W, F), lambda b: (b, 0, 0)),
            pl.BlockSpec((W, F), lambda b: (0, 0)),
        ],
        out_specs=pl.BlockSpec((BB, W, F), lambda b: (b, 0, 0)),
        out_shape=jax.ShapeDtypeStruct(x.shape, x.dtype),
        compiler_params=pltpu.CompilerParams(
            vmem_limit_bytes=100 * 1024 * 1024,
            dimension_semantics=("parallel",),
        ),
    )(x, emb)
